# baseline (speedup 1.0000x reference)
import functools

import jax
import jax.numpy as jnp
from jax.experimental import pallas as pl
from jax.experimental.pallas import tpu as pltpu

M_BLOCK = 512
H_BLOCK = 1024


def _gating_kernel(n_h, x_ref, w1_ref, b1_ref, w2_ref, b2_ref, out_ref):
    h_idx = pl.program_id(1)
    h = jax.lax.dot_general(
        x_ref[...], w1_ref[...], (((1,), (0,)), ((), ())),
        preferred_element_type=jnp.float32)
    h = jnp.maximum(h + b1_ref[...], 0.0)
    part = jax.lax.dot_general(
        h, w2_ref[...], (((1,), (0,)), ((), ())),
        preferred_element_type=jnp.float32)

    @pl.when(h_idx == 0)
    def _init():
        out_ref[...] = part

    @pl.when(h_idx != 0)
    def _acc():
        out_ref[...] += part

    @pl.when(h_idx == n_h - 1)
    def _softmax():
        logits = out_ref[...] + b2_ref[...]
        mx = jnp.max(logits, axis=-1, keepdims=True)
        e = jnp.exp(logits - mx)
        out_ref[...] = e / jnp.sum(e, axis=-1, keepdims=True)


def kernel(inputs, W1, b1, W2, b2):
    M, K = inputs.shape
    H = W1.shape[1]
    E = W2.shape[1]
    n_m = M // M_BLOCK
    n_h = H // H_BLOCK
    return pl.pallas_call(
        functools.partial(_gating_kernel, n_h),
        grid=(n_m, n_h),
        in_specs=[
            pl.BlockSpec((M_BLOCK, K), lambda m, h: (m, 0)),
            pl.BlockSpec((K, H_BLOCK), lambda m, h: (0, h)),
            pl.BlockSpec((1, H_BLOCK), lambda m, h: (0, h)),
            pl.BlockSpec((H_BLOCK, E), lambda m, h: (h, 0)),
            pl.BlockSpec((1, E), lambda m, h: (0, 0)),
        ],
        out_specs=pl.BlockSpec((M_BLOCK, E), lambda m, h: (m, 0)),
        out_shape=jax.ShapeDtypeStruct((M, E), jnp.float32),
        compiler_params=pltpu.CompilerParams(
            dimension_semantics=("parallel", "arbitrary"),
        ),
    )(inputs, W1, b1.reshape(1, H), W2, b2.reshape(1, E))
